# Initial kernel scaffold; baseline (speedup 1.0000x reference)
import jax
import jax.numpy as jnp
from jax.experimental import pallas as pl


def kernel(x, edge_index, Wg1, bg1, Wg2, bg2, W1, b1, W2, b2, W3, b3, W4, b4):
    raise NotImplementedError("write your pallas kernel here")


# speedup vs baseline: 13.4704x; 13.4704x over previous
import functools

import jax
import jax.numpy as jnp
from jax import lax
from jax.experimental import pallas as pl
from jax.experimental.pallas import tpu as pltpu
from jax.experimental.pallas import tpu_sc as plsc

_LANES = 16


def _sc_mesh():
    return plsc.VectorSubcoreMesh(core_axis_name="c", subcore_axis_name="s")


@functools.lru_cache(maxsize=None)
def _make_deg_kernel(n, e, nc, ns):
    nw = nc * ns
    per = e // nw
    b = 128
    full = per // b
    tail = per - full * b
    rpt = (n // ns) // 8 * 8
    extra = n - ns * rpt

    @functools.partial(
        pl.kernel,
        out_type=jax.ShapeDtypeStruct((nc, n, _LANES), jnp.float32),
        mesh=_sc_mesh(),
        scratch_types=[
            pltpu.VMEM((b,), jnp.int32),
            pltpu.VMEM((max(tail, 8),), jnp.int32),
            pltpu.VMEM((b, _LANES), jnp.float32),
            pltpu.VMEM_SHARED((n, _LANES), jnp.float32),
        ],
    )
    def deg_kernel(dst_h, out_h, didx, tdidx, buf, acc):
        c = lax.axis_index("c")
        s = lax.axis_index("s")
        wid = s * nc + c

        zero16 = jnp.zeros((_LANES,), jnp.float32)

        def zrow(r, _):
            buf[r, :] = zero16
            return 0

        lax.fori_loop(0, b, zrow, 0, unroll=8)

        nz = rpt // b + (1 if rpt % b else 0)
        for q in range(nz):
            lo = q * b
            sz = min(b, rpt - lo)
            pltpu.sync_copy(
                buf.at[pl.ds(0, sz)], acc.at[pl.ds(s * rpt + lo, sz)]
            )
        if extra:
            @pl.when(s == 0)
            def _zx():
                pltpu.sync_copy(
                    buf.at[pl.ds(0, extra)], acc.at[pl.ds(ns * rpt, extra)]
                )

        one16 = jnp.where(
            lax.iota(jnp.int32, _LANES) == 0, 1.0, 0.0
        ).astype(jnp.float32)

        def orow(r, _):
            buf[r, :] = one16
            return 0

        lax.fori_loop(0, b, orow, 0, unroll=8)
        plsc.subcore_barrier()

        def step(i, _):
            base = wid * per + i * b
            pltpu.sync_copy(dst_h.at[pl.ds(base, b)], didx)
            pltpu.sync_copy(buf, acc.at[didx], add=True)
            return 0

        lax.fori_loop(0, full, step, 0)
        if tail:
            tbase = wid * per + full * b
            pltpu.sync_copy(dst_h.at[pl.ds(tbase, tail)], tdidx.at[pl.ds(0, tail)])
            pltpu.sync_copy(
                buf.at[pl.ds(0, tail)], acc.at[tdidx.at[pl.ds(0, tail)]], add=True
            )
        plsc.subcore_barrier()

        pltpu.sync_copy(
            acc.at[pl.ds(s * rpt, rpt)], out_h.at[c, pl.ds(s * rpt, rpt)]
        )
        if extra:
            @pl.when(s == 0)
            def _rx():
                pltpu.sync_copy(
                    acc.at[pl.ds(ns * rpt, extra)],
                    out_h.at[c, pl.ds(ns * rpt, extra)],
                )

    return deg_kernel


@functools.lru_cache(maxsize=None)
def _make_agg_kernel(n, e, d, nc, ns):
    nw = nc * ns
    per = e // nw
    b = 128
    full = per // b
    tail = per - full * b
    rpt = (n // ns) // 8 * 8
    extra = n - ns * rpt

    @functools.partial(
        pl.kernel,
        out_type=jax.ShapeDtypeStruct((nc, n, d), jnp.float32),
        mesh=_sc_mesh(),
        scratch_types=[
            pltpu.VMEM((b,), jnp.int32),
            pltpu.VMEM((b,), jnp.int32),
            pltpu.VMEM((max(tail, 8),), jnp.int32),
            pltpu.VMEM((max(tail, 8),), jnp.int32),
            pltpu.VMEM((b, d), jnp.float32),
            pltpu.VMEM_SHARED((n, d), jnp.float32),
            pltpu.SemaphoreType.DMA,
        ],
    )
    def agg_kernel(g_h, src_h, dst_h, out_h, sidx, didx, tsidx, tdidx, rows, acc, sem):
        c = lax.axis_index("c")
        s = lax.axis_index("s")
        wid = s * nc + c

        zero16 = jnp.zeros((_LANES,), jnp.float32)

        def zrow(r, _):
            for k in range(d // _LANES):
                rows[r, pl.ds(k * _LANES, _LANES)] = zero16
            return 0

        lax.fori_loop(0, b, zrow, 0)

        nz = rpt // b + (1 if rpt % b else 0)
        for q in range(nz):
            lo = q * b
            sz = min(b, rpt - lo)
            pltpu.sync_copy(
                rows.at[pl.ds(0, sz)], acc.at[pl.ds(s * rpt + lo, sz)]
            )
        if extra:
            @pl.when(s == 0)
            def _zx():
                pltpu.sync_copy(
                    rows.at[pl.ds(0, extra)], acc.at[pl.ds(ns * rpt, extra)]
                )
        plsc.subcore_barrier()

        def step(i, _):
            base = wid * per + i * b
            pltpu.sync_copy(src_h.at[pl.ds(base, b)], sidx)
            pltpu.sync_copy(dst_h.at[pl.ds(base, b)], didx)
            pltpu.async_copy(g_h.at[sidx], rows, sem).wait()
            pltpu.sync_copy(rows, acc.at[didx], add=True)
            return 0

        lax.fori_loop(0, full, step, 0)
        if tail:
            tbase = wid * per + full * b
            pltpu.sync_copy(src_h.at[pl.ds(tbase, tail)], tsidx.at[pl.ds(0, tail)])
            pltpu.sync_copy(dst_h.at[pl.ds(tbase, tail)], tdidx.at[pl.ds(0, tail)])
            pltpu.async_copy(
                g_h.at[tsidx.at[pl.ds(0, tail)]], rows.at[pl.ds(0, tail)], sem
            ).wait()
            pltpu.sync_copy(
                rows.at[pl.ds(0, tail)], acc.at[tdidx.at[pl.ds(0, tail)]], add=True
            )
        plsc.subcore_barrier()

        pltpu.sync_copy(
            acc.at[pl.ds(s * rpt, rpt)], out_h.at[c, pl.ds(s * rpt, rpt)]
        )
        if extra:
            @pl.when(s == 0)
            def _rx():
                pltpu.sync_copy(
                    acc.at[pl.ds(ns * rpt, extra)],
                    out_h.at[c, pl.ds(ns * rpt, extra)],
                )

    return agg_kernel


def _dinv_of(degp_ref):
    deg = degp_ref[0, :, 0:1] + degp_ref[1, :, 0:1] + 1.0
    return lax.rsqrt(deg)


def _tc1_body(x_ref, w_ref, degp_ref, o_ref):
    dinv = _dinv_of(degp_ref)
    h = jnp.dot(x_ref[...], w_ref[...], preferred_element_type=jnp.float32)
    o_ref[...] = h * dinv


def _tc2_body(degp_ref, p_ref, g_ref, bg_ref, w_ref, o_ref):
    dinv = _dinv_of(degp_ref)
    agg = (p_ref[0] + p_ref[1] + g_ref[...]) * dinv + bg_ref[...]
    h = jnp.maximum(agg, 0.0)
    h = jnp.dot(h, w_ref[...], preferred_element_type=jnp.float32)
    o_ref[...] = h * dinv


def _elu(v):
    return jnp.where(v > 0.0, v, jnp.exp(jnp.minimum(v, 0.0)) - 1.0)


def _tc3_body(degp_ref, p_ref, g_ref, bg_ref, w1, b1, w2, b2, w3, b3, w4, b4, o_ref):
    dinv = _dinv_of(degp_ref)
    h = (p_ref[0] + p_ref[1] + g_ref[...]) * dinv + bg_ref[...]
    h = _elu(jnp.dot(h, w1[...], preferred_element_type=jnp.float32) + b1[...])
    h = _elu(jnp.dot(h, w2[...], preferred_element_type=jnp.float32) + b2[...])
    h = _elu(jnp.dot(h, w3[...], preferred_element_type=jnp.float32) + b3[...])
    h = _elu(jnp.dot(h, w4[...], preferred_element_type=jnp.float32) + b4[...])
    o_ref[...] = 1.0 / (1.0 + jnp.exp(-h))


def _row_spec(blk, d):
    return pl.BlockSpec((blk, d), lambda i: (i, 0))


def _full_spec(shape):
    nd = len(shape)
    return pl.BlockSpec(shape, lambda i: (0,) * nd)


def _degp_spec(blk):
    return pl.BlockSpec((2, blk, _LANES), lambda i: (0, i, 0))


def _part_spec(blk, d):
    return pl.BlockSpec((2, blk, d), lambda i: (0, i, 0))


def kernel(x, edge_index, Wg1, bg1, Wg2, bg2, W1, b1, W2, b2, W3, b3, W4, b4):
    n, d = x.shape
    e = edge_index.shape[1]
    info = plsc.get_sparse_core_info()
    nc, ns = info.num_cores, info.num_subcores

    src = edge_index[0]
    dst = edge_index[1]

    degp = _make_deg_kernel(n, e, nc, ns)(dst)

    blk = 1000
    grid = (n // blk,)

    g1 = pl.pallas_call(
        _tc1_body,
        grid=grid,
        in_specs=[
            _row_spec(blk, d),
            _full_spec((d, d)),
            _degp_spec(blk),
        ],
        out_specs=_row_spec(blk, d),
        out_shape=jax.ShapeDtypeStruct((n, d), jnp.float32),
    )(x, Wg1, degp)

    agg = _make_agg_kernel(n, e, d, nc, ns)

    p1 = agg(g1, src, dst)

    g2 = pl.pallas_call(
        _tc2_body,
        grid=grid,
        in_specs=[
            _degp_spec(blk),
            _part_spec(blk, d),
            _row_spec(blk, d),
            _full_spec((1, d)),
            _full_spec((d, d)),
        ],
        out_specs=_row_spec(blk, d),
        out_shape=jax.ShapeDtypeStruct((n, d), jnp.float32),
    )(degp, p1, g1, bg1[None, :], Wg2)

    p2 = agg(g2, src, dst)

    w4p = jnp.pad(W4, ((0, 0), (0, d - W4.shape[1])))
    b4p = jnp.pad(b4, (0, d - b4.shape[0]))

    out = pl.pallas_call(
        _tc3_body,
        grid=grid,
        in_specs=[
            _degp_spec(blk),
            _part_spec(blk, d),
            _row_spec(blk, d),
            _full_spec((1, d)),
            _full_spec((d, d)),
            _full_spec((1, d)),
            _full_spec((d, d)),
            _full_spec((1, d)),
            _full_spec((d, d)),
            _full_spec((1, d)),
            _full_spec((d, d)),
            _full_spec((1, d)),
        ],
        out_specs=_row_spec(blk, d),
        out_shape=jax.ShapeDtypeStruct((n, d), jnp.float32),
    )(degp, p2, g2, bg2[None, :], W1, b1[None, :], W2, b2[None, :],
      W3, b3[None, :], w4p, b4p[None, :])

    return out[:, :1]


# speedup vs baseline: 26.4718x; 1.9652x over previous
import functools

import jax
import jax.numpy as jnp
from jax import lax
from jax.experimental import pallas as pl
from jax.experimental.pallas import tpu as pltpu
from jax.experimental.pallas import tpu_sc as plsc

_LANES = 16
_B = 128
_NBUF = 4


def _sc_mesh():
    return plsc.VectorSubcoreMesh(core_axis_name="c", subcore_axis_name="s")


@functools.lru_cache(maxsize=None)
def _make_deg_kernel(n_pad, e_pad, nc, ns):
    nw = nc * ns
    chunks = e_pad // (nw * _B)
    rpt = n_pad // ns
    throttle = 8

    @functools.partial(
        pl.kernel,
        out_type=jax.ShapeDtypeStruct((nc, n_pad, _LANES), jnp.float32),
        mesh=_sc_mesh(),
        scratch_types=[
            [pltpu.VMEM((_B,), jnp.int32)] * 4,
            pltpu.VMEM((_B, _LANES), jnp.float32),
            pltpu.VMEM_SHARED((n_pad, _LANES), jnp.float32),
            [pltpu.SemaphoreType.DMA] * 4,
        ],
    )
    def deg_kernel(dst_h, out_h, didx, buf, acc, isem):
        c = lax.axis_index("c")
        s = lax.axis_index("s")
        wid = s * nc + c
        cbase = wid * chunks

        zero16 = jnp.zeros((_LANES,), jnp.float32)

        def zrow(r, _):
            buf[r, :] = zero16
            return 0

        lax.fori_loop(0, _B, zrow, 0, unroll=8)

        for q in range(rpt // _B):
            pltpu.sync_copy(buf, acc.at[pl.ds(s * rpt + q * _B, _B)])

        one16 = jnp.where(lax.iota(jnp.int32, _LANES) == 0, 1.0, 0.0).astype(
            jnp.float32
        )

        def orow(r, _):
            buf[r, :] = one16
            return 0

        lax.fori_loop(0, _B, orow, 0, unroll=8)
        plsc.subcore_barrier()

        def idx_load(i, j):
            pltpu.async_copy(dst_h.at[cbase + i], didx[j], isem[j])

        def wait_idx(j):
            pltpu.make_async_copy(dst_h.at[0], didx[j], isem[j]).wait()

        for k in range(3):
            idx_load(k, k)

        def group(g, _):
            base = g * 4
            for j in range(4):
                i = base + j
                idx_load(i + 3, (j + 3) % 4)
                wait_idx(j)
                pltpu.sync_copy(buf, acc.at[didx[j]], add=True)
            return 0

        lax.fori_loop(0, chunks // 4 - 1, group, 0)

        for j in range(4):
            i = chunks - 4 + j
            if i + 3 < chunks:
                idx_load(i + 3, (j + 3) % 4)
            wait_idx(j)
            pltpu.sync_copy(buf, acc.at[didx[j]], add=True)
        plsc.subcore_barrier()

        pltpu.sync_copy(
            acc.at[pl.ds(s * rpt, rpt)], out_h.at[c, pl.ds(s * rpt, rpt)]
        )

    return deg_kernel


@functools.lru_cache(maxsize=None)
def _make_agg_kernel(n_pad, e_pad, d, nc, ns):
    nw = nc * ns
    dh = d
    chunks = e_pad // (nw * _B)
    rpt = n_pad // ns
    assert chunks % _NBUF == 0 and chunks >= 2 * _NBUF

    @functools.partial(
        pl.kernel,
        out_type=jax.ShapeDtypeStruct((nc, n_pad, dh), jnp.float32),
        mesh=_sc_mesh(),
        scratch_types=[
            [pltpu.VMEM((_B,), jnp.int32)] * 4,
            [pltpu.VMEM((_B,), jnp.int32)] * 4,
            [pltpu.VMEM((_B, dh), jnp.float32)] * 2,
            pltpu.VMEM_SHARED((n_pad, dh), jnp.float32),
            [pltpu.SemaphoreType.DMA] * 4,
            [pltpu.SemaphoreType.DMA] * 2,
        ],
    )
    def agg_kernel(g_h, src_h, dst_h, out_h, sidx, didx, rows, acc, isem,
                   gsem):
        c = lax.axis_index("c")
        s = lax.axis_index("s")
        wid = s * nc + c
        cbase = wid * chunks

        zero16 = jnp.zeros((_LANES,), jnp.float32)

        def zrow(r, _):
            for k in range(dh // _LANES):
                rows[0][r, pl.ds(k * _LANES, _LANES)] = zero16
            return 0

        lax.fori_loop(0, _B, zrow, 0)

        for q in range(rpt // _B):
            pltpu.sync_copy(rows[0], acc.at[pl.ds(s * rpt + q * _B, _B)])
        plsc.subcore_barrier()

        def idx_load(i, j):
            pltpu.async_copy(src_h.at[cbase + i], sidx[j], isem[j])
            pltpu.async_copy(dst_h.at[cbase + i], didx[j], isem[j])

        def wait_idx(j):
            pltpu.make_async_copy(src_h.at[0], sidx[j], isem[j]).wait()
            pltpu.make_async_copy(dst_h.at[0], didx[j], isem[j]).wait()

        def gather(jr, j4):
            pltpu.async_copy(g_h.at[sidx[j4]], rows[jr], gsem[jr])

        def wait_gather(jr, j4):
            pltpu.make_async_copy(g_h.at[sidx[j4]], rows[jr],
                                  gsem[jr]).wait()

        def scatter(jr, j4):
            pltpu.sync_copy(rows[jr], acc.at[didx[j4]], add=True)

        for k in range(3):
            idx_load(k, k)
        wait_idx(0)
        gather(0, 0)

        def group(g, _):
            base = g * 4
            for j in range(4):
                i = base + j
                idx_load(i + 3, (j + 3) % 4)
                wait_idx((j + 1) % 4)
                gather((j + 1) % 2, (j + 1) % 4)
                wait_gather(j % 2, j)
                scatter(j % 2, j)
            return 0

        lax.fori_loop(0, chunks // 4 - 1, group, 0)

        base = chunks - 4
        for j in range(4):
            i = base + j
            if i + 3 < chunks:
                idx_load(i + 3, (j + 3) % 4)
            if i + 1 < chunks:
                wait_idx((j + 1) % 4)
                gather((j + 1) % 2, (j + 1) % 4)
            wait_gather(j % 2, j)
            scatter(j % 2, j)
        plsc.subcore_barrier()

        pltpu.sync_copy(
            acc.at[pl.ds(s * rpt, rpt)], out_h.at[c, pl.ds(s * rpt, rpt)]
        )

    return agg_kernel


def _dinv_of(degp_ref):
    deg = degp_ref[0, :, 0:1] + degp_ref[1, :, 0:1] + 1.0
    return lax.rsqrt(deg)


def _tc1_body(x_ref, w_ref, degp_ref, o_ref):
    dinv = _dinv_of(degp_ref)
    h = jnp.dot(x_ref[...], w_ref[...], preferred_element_type=jnp.float32)
    o_ref[...] = h * dinv


def _tc2_body(degp_ref, p_ref, g_ref, bg_ref, w_ref, o_ref):
    dinv = _dinv_of(degp_ref)
    agg = (p_ref[0] + p_ref[1] + g_ref[...]) * dinv + bg_ref[...]
    h = jnp.maximum(agg, 0.0)
    h = jnp.dot(h, w_ref[...], preferred_element_type=jnp.float32)
    o_ref[...] = h * dinv


def _elu(v):
    return jnp.where(v > 0.0, v, jnp.exp(jnp.minimum(v, 0.0)) - 1.0)


def _tc3_body(degp_ref, p_ref, g_ref, bg_ref, w1, b1, w2, b2, w3, b3, w4, b4,
              o_ref):
    dinv = _dinv_of(degp_ref)
    h = (p_ref[0] + p_ref[1] + g_ref[...]) * dinv + bg_ref[...]
    h = _elu(jnp.dot(h, w1[...], preferred_element_type=jnp.float32) + b1[...])
    h = _elu(jnp.dot(h, w2[...], preferred_element_type=jnp.float32) + b2[...])
    h = _elu(jnp.dot(h, w3[...], preferred_element_type=jnp.float32) + b3[...])
    h = _elu(jnp.dot(h, w4[...], preferred_element_type=jnp.float32) + b4[...])
    o_ref[...] = 1.0 / (1.0 + jnp.exp(-h))


def _row_spec(blk, d):
    return pl.BlockSpec((blk, d), lambda i: (i, 0))


def _full_spec(shape):
    nd = len(shape)
    return pl.BlockSpec(shape, lambda i: (0,) * nd)


def _degp_spec(blk):
    return pl.BlockSpec((2, blk, _LANES), lambda i: (0, i, 0))


def _part_spec(blk, dh):
    return pl.BlockSpec((2, blk, dh), lambda i: (0, i, 0))


def kernel(x, edge_index, Wg1, bg1, Wg2, bg2, W1, b1, W2, b2, W3, b3, W4, b4):
    n, d = x.shape
    e = edge_index.shape[1]
    info = plsc.get_sparse_core_info()
    nc, ns = info.num_cores, info.num_subcores
    nw = nc * ns

    blk = 1024
    n_pad = -(-n // blk) * blk
    grid = (n_pad // blk,)
    cpt = -(-e // (nw * _B))
    cpt = -(-cpt // _NBUF) * _NBUF
    e_pad = cpt * nw * _B

    src = edge_index[0]
    dst = edge_index[1]
    pad = n + (jnp.arange(e_pad - e, dtype=jnp.int32) % (n_pad - n))
    src_p2 = jnp.concatenate([src, pad]).reshape(e_pad // _B, _B)
    dst_p2 = jnp.concatenate([dst, pad]).reshape(e_pad // _B, _B)
    x_p = jnp.pad(x, ((0, n_pad - n), (0, 0)))
    dh = d

    degp = _make_deg_kernel(n_pad, e_pad, nc, ns)(dst_p2)

    g1 = pl.pallas_call(
        _tc1_body,
        grid=grid,
        in_specs=[
            _row_spec(blk, d),
            _full_spec((d, d)),
            _degp_spec(blk),
        ],
        out_specs=_row_spec(blk, d),
        out_shape=jax.ShapeDtypeStruct((n_pad, d), jnp.float32),
    )(x_p, Wg1, degp)

    agg = _make_agg_kernel(n_pad, e_pad, d, nc, ns)

    p1 = agg(g1, src_p2, dst_p2)

    g2 = pl.pallas_call(
        _tc2_body,
        grid=grid,
        in_specs=[
            _degp_spec(blk),
            _part_spec(blk, dh),
            _row_spec(blk, d),
            _full_spec((1, d)),
            _full_spec((d, d)),
        ],
        out_specs=_row_spec(blk, d),
        out_shape=jax.ShapeDtypeStruct((n_pad, d), jnp.float32),
    )(degp, p1, g1, bg1[None, :], Wg2)

    p2 = agg(g2, src_p2, dst_p2)

    w4p = jnp.pad(W4, ((0, 0), (0, d - W4.shape[1])))
    b4p = jnp.pad(b4, (0, d - b4.shape[0]))

    out = pl.pallas_call(
        _tc3_body,
        grid=grid,
        in_specs=[
            _degp_spec(blk),
            _part_spec(blk, dh),
            _row_spec(blk, d),
            _full_spec((1, d)),
            _full_spec((d, d)),
            _full_spec((1, d)),
            _full_spec((d, d)),
            _full_spec((1, d)),
            _full_spec((d, d)),
            _full_spec((1, d)),
            _full_spec((d, d)),
            _full_spec((1, d)),
        ],
        out_specs=_row_spec(blk, d),
        out_shape=jax.ShapeDtypeStruct((n_pad, d), jnp.float32),
    )(degp, p2, g2, bg2[None, :], W1, b1[None, :], W2, b2[None, :],
      W3, b3[None, :], w4p, b4p[None, :])

    return out[:n, :1]


# speedup vs baseline: 29.1016x; 1.0993x over previous
import functools

import jax
import jax.numpy as jnp
from jax import lax
from jax.experimental import pallas as pl
from jax.experimental.pallas import tpu as pltpu
from jax.experimental.pallas import tpu_sc as plsc

_LANES = 16
_B = 128
_BA = 120
_NBUF = 4


def _sc_mesh():
    return plsc.VectorSubcoreMesh(core_axis_name="c", subcore_axis_name="s")


@functools.lru_cache(maxsize=None)
def _make_deg_kernel(n_pad, e_pad, nc, ns):
    nw = nc * ns
    chunks = e_pad // (nw * _B)
    rpt = n_pad // ns
    throttle = 8

    @functools.partial(
        pl.kernel,
        out_type=jax.ShapeDtypeStruct((nc, n_pad, _LANES), jnp.float32),
        mesh=_sc_mesh(),
        scratch_types=[
            [pltpu.VMEM((_B,), jnp.int32)] * 4,
            pltpu.VMEM((_B, _LANES), jnp.float32),
            pltpu.VMEM_SHARED((n_pad, _LANES), jnp.float32),
            [pltpu.SemaphoreType.DMA] * 4,
        ],
    )
    def deg_kernel(dst_h, out_h, didx, buf, acc, isem):
        c = lax.axis_index("c")
        s = lax.axis_index("s")
        wid = s * nc + c
        cbase = wid * chunks

        zero16 = jnp.zeros((_LANES,), jnp.float32)

        def zrow(r, _):
            buf[r, :] = zero16
            return 0

        lax.fori_loop(0, _B, zrow, 0, unroll=8)

        for q in range(rpt // _B):
            pltpu.sync_copy(buf, acc.at[pl.ds(s * rpt + q * _B, _B)])

        one16 = jnp.where(lax.iota(jnp.int32, _LANES) == 0, 1.0, 0.0).astype(
            jnp.float32
        )

        def orow(r, _):
            buf[r, :] = one16
            return 0

        lax.fori_loop(0, _B, orow, 0, unroll=8)
        plsc.subcore_barrier()

        def idx_load(i, j):
            pltpu.async_copy(dst_h.at[cbase + i], didx[j], isem[j])

        def wait_idx(j):
            pltpu.make_async_copy(dst_h.at[0], didx[j], isem[j]).wait()

        for k in range(3):
            idx_load(k, k)

        def group(g, _):
            base = g * 4
            for j in range(4):
                i = base + j
                idx_load(i + 3, (j + 3) % 4)
                wait_idx(j)
                pltpu.sync_copy(buf, acc.at[didx[j]], add=True)
            return 0

        lax.fori_loop(0, chunks // 4 - 1, group, 0)

        for j in range(4):
            i = chunks - 4 + j
            if i + 3 < chunks:
                idx_load(i + 3, (j + 3) % 4)
            wait_idx(j)
            pltpu.sync_copy(buf, acc.at[didx[j]], add=True)
        plsc.subcore_barrier()

        pltpu.sync_copy(
            acc.at[pl.ds(s * rpt, rpt)], out_h.at[c, pl.ds(s * rpt, rpt)]
        )

    return deg_kernel


@functools.lru_cache(maxsize=None)
def _make_agg_kernel(n_pad, e_pad, d, nc, ns):
    nw = nc * ns
    dh = d
    chunks = e_pad // (nw * _BA)
    rpt = n_pad // ns
    assert chunks % 6 == 0 and chunks >= 12

    @functools.partial(
        pl.kernel,
        out_type=jax.ShapeDtypeStruct((nc, n_pad, dh), jnp.float32),
        mesh=_sc_mesh(),
        scratch_types=[
            [pltpu.VMEM((_BA,), jnp.int32)] * 6,
            [pltpu.VMEM((_BA,), jnp.int32)] * 6,
            [pltpu.VMEM((_BA, dh), jnp.float32)] * 3,
            pltpu.VMEM_SHARED((n_pad, dh), jnp.float32),
            [pltpu.SemaphoreType.DMA] * 6,
            [pltpu.SemaphoreType.DMA] * 3,
            [pltpu.SemaphoreType.DMA] * 3,
        ],
    )
    def agg_kernel(g_h, src_h, dst_h, out_h, sidx, didx, rows, acc, isem,
                   gsem, ssem):
        c = lax.axis_index("c")
        s = lax.axis_index("s")
        wid = s * nc + c
        cbase = wid * chunks

        zero16 = jnp.zeros((_LANES,), jnp.float32)

        def zrow(r, _):
            for k in range(dh // _LANES):
                rows[0][r, pl.ds(k * _LANES, _LANES)] = zero16
            return 0

        lax.fori_loop(0, _BA, zrow, 0)

        nfull = rpt // _BA
        for q in range(nfull):
            pltpu.sync_copy(rows[0].at[pl.ds(0, _BA)],
                            acc.at[pl.ds(s * rpt + q * _BA, _BA)])
        rem = rpt - nfull * _BA
        if rem:
            pltpu.sync_copy(rows[0].at[pl.ds(0, rem)],
                            acc.at[pl.ds(s * rpt + nfull * _BA, rem)])
        plsc.subcore_barrier()

        def idx_load(i, j):
            pltpu.async_copy(src_h.at[cbase + i], sidx[j], isem[j])
            pltpu.async_copy(dst_h.at[cbase + i], didx[j], isem[j])

        def wait_idx(j):
            pltpu.make_async_copy(src_h.at[0], sidx[j], isem[j]).wait()
            pltpu.make_async_copy(dst_h.at[0], didx[j], isem[j]).wait()

        def gather(jr, j6):
            pltpu.async_copy(g_h.at[sidx[j6]], rows[jr], gsem[jr])

        def wait_gather(jr, j6):
            pltpu.make_async_copy(g_h.at[sidx[j6]], rows[jr],
                                  gsem[jr]).wait()

        def scatter(jr, j6):
            pltpu.sync_copy(rows[jr], acc.at[didx[j6]], add=True)

        def wait_scatter(jr):
            pass

        def step(i, j, last_grp=False):
            if not last_grp or j < 2:
                idx_load(i + 4, (j + 4) % 6)
            if not last_grp or j < 4:
                wait_idx((j + 2) % 6)
                gather((j + 2) % 3, (j + 2) % 6)
            wait_gather(j % 3, j)
            scatter(j % 3, j)

        for k in range(4):
            idx_load(k, k)
        for k in range(2):
            wait_idx(k)
            gather(k, k)

        for j in range(6):
            step(j, j)

        def group(g, _):
            base = g * 6
            for j in range(6):
                step(base + j, j)
            return 0

        lax.fori_loop(1, chunks // 6 - 1, group, 0)

        base = chunks - 6
        for j in range(6):
            step(base + j, j, last_grp=True)
        plsc.subcore_barrier()

        pltpu.sync_copy(
            acc.at[pl.ds(s * rpt, rpt)], out_h.at[c, pl.ds(s * rpt, rpt)]
        )

    return agg_kernel


def _dinv_of(degp_ref):
    deg = degp_ref[0, :, 0:1] + degp_ref[1, :, 0:1] + 1.0
    return lax.rsqrt(deg)


def _tc1_body(x_ref, w_ref, degp_ref, o_ref):
    dinv = _dinv_of(degp_ref)
    h = jnp.dot(x_ref[...], w_ref[...], preferred_element_type=jnp.float32)
    o_ref[...] = h * dinv


def _tc2_body(degp_ref, p_ref, g_ref, bg_ref, w_ref, o_ref):
    dinv = _dinv_of(degp_ref)
    agg = (p_ref[0] + p_ref[1] + g_ref[...]) * dinv + bg_ref[...]
    h = jnp.maximum(agg, 0.0)
    h = jnp.dot(h, w_ref[...], preferred_element_type=jnp.float32)
    o_ref[...] = h * dinv


def _elu(v):
    return jnp.where(v > 0.0, v, jnp.exp(jnp.minimum(v, 0.0)) - 1.0)


def _tc3_body(degp_ref, p_ref, g_ref, bg_ref, w1, b1, w2, b2, w3, b3, w4, b4,
              o_ref):
    dinv = _dinv_of(degp_ref)
    h = (p_ref[0] + p_ref[1] + g_ref[...]) * dinv + bg_ref[...]
    h = _elu(jnp.dot(h, w1[...], preferred_element_type=jnp.float32) + b1[...])
    h = _elu(jnp.dot(h, w2[...], preferred_element_type=jnp.float32) + b2[...])
    h = _elu(jnp.dot(h, w3[...], preferred_element_type=jnp.float32) + b3[...])
    h = _elu(jnp.dot(h, w4[...], preferred_element_type=jnp.float32) + b4[...])
    o_ref[...] = 1.0 / (1.0 + jnp.exp(-h))


def _row_spec(blk, d):
    return pl.BlockSpec((blk, d), lambda i: (i, 0))


def _full_spec(shape):
    nd = len(shape)
    return pl.BlockSpec(shape, lambda i: (0,) * nd)


def _degp_spec(blk):
    return pl.BlockSpec((2, blk, _LANES), lambda i: (0, i, 0))


def _part_spec(blk, dh):
    return pl.BlockSpec((2, blk, dh), lambda i: (0, i, 0))


def kernel(x, edge_index, Wg1, bg1, Wg2, bg2, W1, b1, W2, b2, W3, b3, W4, b4):
    n, d = x.shape
    e = edge_index.shape[1]
    info = plsc.get_sparse_core_info()
    nc, ns = info.num_cores, info.num_subcores
    nw = nc * ns

    blk = 1024
    n_pad = -(-n // blk) * blk
    grid = (n_pad // blk,)
    cpt = -(-e // (nw * _B))
    cpt = -(-cpt // _NBUF) * _NBUF
    e_pad = cpt * nw * _B
    cpa = -(-e // (nw * _BA))
    cpa = -(-cpa // 6) * 6
    e_pad_a = cpa * nw * _BA

    src = edge_index[0]
    dst = edge_index[1]
    pad = n + (jnp.arange(e_pad - e, dtype=jnp.int32) % (n_pad - n))
    pad_a = n + (jnp.arange(e_pad_a - e, dtype=jnp.int32) % (n_pad - n))
    dst_p2 = jnp.concatenate([dst, pad]).reshape(e_pad // _B, _B)
    src_a2 = jnp.concatenate([src, pad_a]).reshape(e_pad_a // _BA, _BA)
    dst_a2 = jnp.concatenate([dst, pad_a]).reshape(e_pad_a // _BA, _BA)
    x_p = jnp.pad(x, ((0, n_pad - n), (0, 0)))
    dh = d
    degp = _make_deg_kernel(n_pad, e_pad, nc, ns)(dst_p2)

    g1 = pl.pallas_call(
        _tc1_body,
        grid=grid,
        in_specs=[
            _row_spec(blk, d),
            _full_spec((d, d)),
            _degp_spec(blk),
        ],
        out_specs=_row_spec(blk, d),
        out_shape=jax.ShapeDtypeStruct((n_pad, d), jnp.float32),
    )(x_p, Wg1, degp)

    agg = _make_agg_kernel(n_pad, e_pad_a, d, nc, ns)

    p1 = agg(g1, src_a2, dst_a2)

    g2 = pl.pallas_call(
        _tc2_body,
        grid=grid,
        in_specs=[
            _degp_spec(blk),
            _part_spec(blk, dh),
            _row_spec(blk, d),
            _full_spec((1, d)),
            _full_spec((d, d)),
        ],
        out_specs=_row_spec(blk, d),
        out_shape=jax.ShapeDtypeStruct((n_pad, d), jnp.float32),
    )(degp, p1, g1, bg1[None, :], Wg2)

    p2 = agg(g2, src_a2, dst_a2)

    w4p = jnp.pad(W4, ((0, 0), (0, d - W4.shape[1])))
    b4p = jnp.pad(b4, (0, d - b4.shape[0]))

    out = pl.pallas_call(
        _tc3_body,
        grid=grid,
        in_specs=[
            _degp_spec(blk),
            _part_spec(blk, dh),
            _row_spec(blk, d),
            _full_spec((1, d)),
            _full_spec((d, d)),
            _full_spec((1, d)),
            _full_spec((d, d)),
            _full_spec((1, d)),
            _full_spec((d, d)),
            _full_spec((1, d)),
            _full_spec((d, d)),
            _full_spec((1, d)),
        ],
        out_specs=_row_spec(blk, d),
        out_shape=jax.ShapeDtypeStruct((n_pad, d), jnp.float32),
    )(degp, p2, g2, bg2[None, :], W1, b1[None, :], W2, b2[None, :],
      W3, b3[None, :], w4p, b4p[None, :])

    return out[:n, :1]
